# baseline (speedup 1.0000x reference)
import functools

import jax
import jax.numpy as jnp
from jax import lax
from jax.experimental import pallas as pl
from jax.experimental.pallas import tpu as pltpu
from jax.experimental.pallas import tpu_sc as plsc

N = 10000
E = 320000
D = 128
C = 10
CP = 16

NC = 2
NS = 16
NW = NC * NS
EPW = E // NW
CHUNK = 400
GROUPS = CHUNK // 16
SUB = 80
NSUB = CHUNK // SUB
OUT_TILES = 10
OUT_ROWS = N // OUT_TILES


def _mm_body(w_ref, x_ref, o_ref):
    o_ref[...] = lax.dot_general(
        w_ref[...], x_ref[...], (((0,), (1,)), ((), ())),
        preferred_element_type=jnp.float32)


def _project(x, wp):
    return pl.pallas_call(
        _mm_body,
        out_shape=jax.ShapeDtypeStruct((CP, N), jnp.float32),
    )(wp, x)


FOLD = N * CP // 128


def _comb_body(p_ref, b_ref, o_ref):
    o_ref[...] = p_ref[0] + p_ref[1] + b_ref[...]


def _combine(partial, b128):
    return pl.pallas_call(
        _comb_body,
        in_specs=[
            pl.BlockSpec((NC, FOLD, 128), lambda: (0, 0, 0)),
            pl.BlockSpec((1, 128), lambda: (0, 0)),
        ],
        out_specs=pl.BlockSpec((FOLD, 128), lambda: (0, 0)),
        out_shape=jax.ShapeDtypeStruct((FOLD, 128), jnp.float32),
    )(partial, b128)


_mesh = plsc.VectorSubcoreMesh(core_axis_name="c", subcore_axis_name="s")


@functools.partial(
    pl.kernel,
    mesh=_mesh,
    compiler_params=pltpu.CompilerParams(
        needs_layout_passes=False, use_tc_tiling_on_sc=False),
    out_type=jax.ShapeDtypeStruct((NC, N, CP), jnp.float32),
    scratch_types=(
        [pltpu.VMEM((C * N,), jnp.float32)]
        + [pltpu.VMEM((CHUNK,), jnp.int32) for _ in range(3)]
        + [pltpu.VMEM((NSUB, SUB), jnp.int32) for _ in range(4)]
        + [pltpu.VMEM((CHUNK,), jnp.float32) for _ in range(3)]
        + [pltpu.VMEM((CHUNK, CP), jnp.float32) for _ in range(2)]
        + [pltpu.VMEM_SHARED((N, CP), jnp.float32)]
        + [pltpu.SemaphoreType.DMA for _ in range(3)]
        + [pltpu.SemaphoreType.DMA for _ in range(3)]
        + [pltpu.SemaphoreType.DMA]
    ),
)
def _sc_scatter(xwt_hbm, ei4_hbm, w_hbm, partial_hbm,
                xwt_v, src0_v, src1_v, src2_v,
                dst0_v, dst1_v, dst2_v, dst3_v,
                w0_v, w1_v, w2_v, msgs0_v, msgs1_v, accum,
                lsem0, lsem1, lsem2, ssem0, ssem1, ssem2, osem):
    c = lax.axis_index("c")
    s = lax.axis_index("s")
    wid = c * NS + s
    src_b = (src0_v, src1_v, src2_v)
    w_b = (w0_v, w1_v, w2_v)
    msgs_b = (msgs0_v, msgs1_v)
    dst_b = (dst0_v, dst1_v, dst2_v, dst3_v)
    lsem = (lsem0, lsem1, lsem2)
    ssem = (ssem0, ssem1, ssem2)

    ebase = wid * EPW
    rbase = wid * (EPW // SUB)
    NCH = EPW // CHUNK

    def load_copies(k, kk):
        b2, b3 = kk % 3, kk % 4
        base = ebase + k * CHUNK
        rb = rbase + k * NSUB
        return [
            pltpu.make_async_copy(ei4_hbm.at[0, rb + j],
                                  src_b[b2].at[pl.ds(j * SUB, SUB)],
                                  lsem[b2])
            for j in range(NSUB)
        ] + [
            pltpu.make_async_copy(w_hbm.at[pl.ds(base, CHUNK)], w_b[b2],
                                  lsem[b2]),
            pltpu.make_async_copy(ei4_hbm.at[1, pl.ds(rb, NSUB)],
                                  dst_b[b3], lsem[b2]),
        ]

    def fire_loads(k, kk):
        for d in load_copies(k, kk):
            d.start()

    def wait_loads(k, kk):
        for d in load_copies(k, kk):
            d.wait()

    def scat_copies(kk):
        b2, b3, b4 = kk % 2, kk % 3, kk % 4
        return [
            pltpu.make_async_copy(msgs_b[b2].at[pl.ds(j * SUB, SUB)],
                                  accum.at[dst_b[b4].at[j]], ssem[b3])
            for j in range(NSUB)
        ]

    def fire_scats(kk):
        b2, b3, b4 = kk % 2, kk % 3, kk % 4
        for j in range(NSUB):
            pltpu.async_copy(msgs_b[b2].at[pl.ds(j * SUB, SUB)],
                             accum.at[dst_b[b4].at[j]], ssem[b3], add=True)

    def wait_scats(kk):
        for d in scat_copies(kk):
            d.wait()

    def compute(kk):
        b2, b3 = kk % 2, kk % 3

        @plsc.parallel_loop(0, GROUPS, 1, unroll=3)
        def group_body(g, _src=src_b[b3], _w=w_b[b3], _m=msgs_b[b2]):
            src16 = _src[pl.ds(g * 16, 16)]
            w16 = _w[pl.ds(g * 16, 16)]
            rows = lax.iota(jnp.int32, 16) + g * 16
            for f in range(C):
                idx = src16 + f * N
                vals = plsc.load_gather(xwt_v, [idx])
                fv = jnp.full((16,), f, jnp.int32)
                plsc.store_scatter(_m, [rows, fv], vals * w16)

    fire_loads(0, 0)

    pltpu.sync_copy(xwt_hbm.at[pl.ds(0, C * N)], xwt_v)

    z16 = jnp.zeros((16,), jnp.float32)

    def zero_body(g, carry):
        rows = lax.iota(jnp.int32, 16) + g * 16
        for f in range(CP):
            fv = jnp.full((16,), f, jnp.int32)
            plsc.store_scatter(msgs0_v, [rows, fv], z16)
            plsc.store_scatter(msgs1_v, [rows, fv], z16)
        return carry

    lax.fori_loop(0, GROUPS, zero_body, 0)

    zbase = s * (N // NS)
    pltpu.sync_copy(msgs0_v, accum.at[pl.ds(zbase, CHUNK)])
    pltpu.sync_copy(msgs0_v.at[pl.ds(0, N // NS - CHUNK)],
                    accum.at[pl.ds(zbase + CHUNK, N // NS - CHUNK)])
    plsc.subcore_barrier()

    dseed = pltpu.make_async_copy(
        ei4_hbm.at[1, pl.ds(rbase + 2 * NSUB, NSUB)], dst_b[3], lsem[2])
    dseed.start()
    dseed.wait()
    for j in range(NSUB):
        pltpu.async_copy(msgs1_v.at[pl.ds(j * SUB, SUB)],
                         accum.at[dst_b[3].at[j]], ssem[2], add=True)

    fire_loads(1, 1)
    fire_loads(2, 2)
    wait_loads(0, 0)
    compute(0)
    fire_scats(0)

    def loop_body(jj, carry):
        for i in range(12):
            k = 12 * jj + 1 + i
            kk = 1 + i
            wait_scats(kk - 2)
            fire_loads(jnp.minimum(k + 2, NCH - 1), kk + 2)
            wait_loads(k, kk)
            compute(kk)
            fire_scats(kk)
        return carry

    lax.fori_loop(0, (NCH - 1) // 12, loop_body, 0)

    wait_loads(NCH - 1, 13)
    wait_loads(NCH - 1, 14)
    wait_scats(11)
    wait_scats(12)
    plsc.subcore_barrier()

    @pl.when(s < OUT_TILES)
    def _copy_out():
        oh = []
        for j in range(OUT_ROWS // 40):
            r0 = s * OUT_ROWS + j * 40
            oh.append(pltpu.async_copy(
                accum.at[pl.ds(r0, 40)],
                partial_hbm.at[c, pl.ds(r0, 40)], osem))
        for h in oh:
            h.wait()


def kernel(x, edge_index, edge_weight, W, b):
    ei4 = edge_index.reshape(2, E // SUB, SUB)
    wp = jnp.pad(W, ((0, 0), (0, CP - C)))
    b128 = jnp.tile(jnp.pad(b, (0, CP - C)), 128 // CP).reshape(1, 128)

    xwt = _project(x, wp).reshape(CP * N)
    partial = _sc_scatter(xwt, ei4, edge_weight)
    comb = _combine(partial.reshape(NC, FOLD, 128), b128)
    return comb.reshape(N, CP)[:, :C]


# speedup vs baseline: 1.0314x; 1.0314x over previous
import functools

import jax
import jax.numpy as jnp
from jax import lax
from jax.experimental import pallas as pl
from jax.experimental.pallas import tpu as pltpu
from jax.experimental.pallas import tpu_sc as plsc

N = 10000
E = 320000
D = 128
C = 10
CP = 16

NC = 2
NS = 16
NW = NC * NS
EPW = E // NW
CHUNK = 400
GROUPS = CHUNK // 16
SUB = 80
NSUB = CHUNK // SUB
OUT_TILES = 10
OUT_ROWS = N // OUT_TILES


def _mm_body(w_ref, x_ref, o_ref):
    o_ref[...] = lax.dot_general(
        w_ref[...], x_ref[...], (((0,), (1,)), ((), ())),
        preferred_element_type=jnp.float32)


def _project(x, wp):
    return pl.pallas_call(
        _mm_body,
        out_shape=jax.ShapeDtypeStruct((CP, N), jnp.float32),
    )(wp, x)


FOLD = N * CP // 128


def _comb_body(p_ref, b_ref, o_ref):
    o_ref[...] = p_ref[0] + p_ref[1] + b_ref[...]


def _combine(partial, b128):
    return pl.pallas_call(
        _comb_body,
        in_specs=[
            pl.BlockSpec((NC, FOLD, 128), lambda: (0, 0, 0)),
            pl.BlockSpec((1, 128), lambda: (0, 0)),
        ],
        out_specs=pl.BlockSpec((FOLD, 128), lambda: (0, 0)),
        out_shape=jax.ShapeDtypeStruct((FOLD, 128), jnp.float32),
    )(partial, b128)


_mesh = plsc.VectorSubcoreMesh(core_axis_name="c", subcore_axis_name="s")


@functools.partial(
    pl.kernel,
    mesh=_mesh,
    compiler_params=pltpu.CompilerParams(
        needs_layout_passes=False, use_tc_tiling_on_sc=False),
    out_type=jax.ShapeDtypeStruct((NC, N, CP), jnp.float32),
    scratch_types=(
        [pltpu.VMEM((C * N,), jnp.float32)]
        + [pltpu.VMEM((CHUNK,), jnp.int32) for _ in range(2)]
        + [pltpu.VMEM((NSUB, SUB), jnp.int32) for _ in range(3)]
        + [pltpu.VMEM((CHUNK,), jnp.float32) for _ in range(2)]
        + [pltpu.VMEM((CHUNK, CP), jnp.float32) for _ in range(2)]
        + [pltpu.VMEM_SHARED((N, CP), jnp.float32)]
        + [pltpu.SemaphoreType.DMA for _ in range(2)]
        + [pltpu.SemaphoreType.DMA for _ in range(3)]
        + [pltpu.SemaphoreType.DMA]
    ),
)
def _sc_scatter(xwt_hbm, ei4_hbm, w_hbm, partial_hbm,
                xwt_v, src0_v, src1_v, dst0_v, dst1_v, dst2_v,
                w0_v, w1_v, msgs0_v, msgs1_v, accum,
                lsem0, lsem1, ssem0, ssem1, ssem2, osem):
    c = lax.axis_index("c")
    s = lax.axis_index("s")
    wid = c * NS + s
    src_b = (src0_v, src1_v)
    w_b = (w0_v, w1_v)
    msgs_b = (msgs0_v, msgs1_v)
    dst_b = (dst0_v, dst1_v, dst2_v)
    lsem = (lsem0, lsem1)
    ssem = (ssem0, ssem1, ssem2)

    ebase = wid * EPW
    rbase = wid * (EPW // SUB)
    NCH = EPW // CHUNK

    def load_copies(k, kk):
        b2, b3 = kk % 2, kk % 3
        base = ebase + k * CHUNK
        rb = rbase + k * NSUB
        return [
            pltpu.make_async_copy(ei4_hbm.at[0, rb + j],
                                  src_b[b2].at[pl.ds(j * SUB, SUB)],
                                  lsem[b2])
            for j in range(NSUB)
        ] + [
            pltpu.make_async_copy(w_hbm.at[pl.ds(base, CHUNK)], w_b[b2],
                                  lsem[b2]),
            pltpu.make_async_copy(ei4_hbm.at[1, pl.ds(rb, NSUB)],
                                  dst_b[b3], lsem[b2]),
        ]

    def fire_loads(k, kk):
        for d in load_copies(k, kk):
            d.start()

    def wait_loads(k, kk):
        for d in load_copies(k, kk):
            d.wait()

    def scat_copies(kk):
        b2, b3 = kk % 2, kk % 3
        return [
            pltpu.make_async_copy(msgs_b[b2].at[pl.ds(j * SUB, SUB)],
                                  accum.at[dst_b[b3].at[j]], ssem[b3])
            for j in range(NSUB)
        ]

    def fire_scats(kk):
        b2, b3 = kk % 2, kk % 3
        for j in range(NSUB):
            pltpu.async_copy(msgs_b[b2].at[pl.ds(j * SUB, SUB)],
                             accum.at[dst_b[b3].at[j]], ssem[b3], add=True)

    def wait_scats(kk):
        for d in scat_copies(kk):
            d.wait()

    def compute(kk):
        b2 = kk % 2

        @plsc.parallel_loop(0, GROUPS, 1, unroll=3)
        def group_body(g, _src=src_b[b2], _w=w_b[b2], _m=msgs_b[b2]):
            src16 = _src[pl.ds(g * 16, 16)]
            w16 = _w[pl.ds(g * 16, 16)]
            rows = lax.iota(jnp.int32, 16) + g * 16
            for f in range(C):
                idx = src16 + f * N
                vals = plsc.load_gather(xwt_v, [idx])
                fv = jnp.full((16,), f, jnp.int32)
                plsc.store_scatter(_m, [rows, fv], vals * w16)

    fire_loads(0, 0)

    pltpu.sync_copy(xwt_hbm.at[pl.ds(0, C * N)], xwt_v)

    z16 = jnp.zeros((16,), jnp.float32)

    def zero_body(g, carry):
        rows = lax.iota(jnp.int32, 16) + g * 16
        for f in range(CP):
            fv = jnp.full((16,), f, jnp.int32)
            plsc.store_scatter(msgs0_v, [rows, fv], z16)
            plsc.store_scatter(msgs1_v, [rows, fv], z16)
        return carry

    lax.fori_loop(0, GROUPS, zero_body, 0)

    zbase = s * (N // NS)
    pltpu.sync_copy(msgs0_v, accum.at[pl.ds(zbase, CHUNK)])
    pltpu.sync_copy(msgs0_v.at[pl.ds(0, N // NS - CHUNK)],
                    accum.at[pl.ds(zbase + CHUNK, N // NS - CHUNK)])
    plsc.subcore_barrier()

    dseed = pltpu.make_async_copy(
        ei4_hbm.at[1, pl.ds(rbase + 2 * NSUB, NSUB)], dst_b[2], lsem[1])
    dseed.start()
    dseed.wait()
    for j in range(NSUB):
        pltpu.async_copy(msgs1_v.at[pl.ds(j * SUB, SUB)],
                         accum.at[dst_b[2].at[j]], ssem[2], add=True)

    fire_loads(1, 1)
    wait_loads(0, 0)
    compute(0)
    fire_scats(0)

    def loop_body(jj, carry):
        for i in range(6):
            k = 6 * jj + 1 + i
            kk = 1 + i
            wait_scats(kk - 2)
            fire_loads(jnp.minimum(k + 1, NCH - 1), kk + 1)
            wait_loads(k, kk)
            compute(kk)
            fire_scats(kk)
        return carry

    lax.fori_loop(0, (NCH - 1) // 6, loop_body, 0)

    wait_loads(NCH - 1, 7)
    wait_scats(5)
    wait_scats(6)
    plsc.subcore_barrier()

    @pl.when(s < OUT_TILES)
    def _copy_out():
        oh = []
        for j in range(OUT_ROWS // 40):
            r0 = s * OUT_ROWS + j * 40
            oh.append(pltpu.async_copy(
                accum.at[pl.ds(r0, 40)],
                partial_hbm.at[c, pl.ds(r0, 40)], osem))
        for h in oh:
            h.wait()


def kernel(x, edge_index, edge_weight, W, b):
    ei4 = edge_index.reshape(2, E // SUB, SUB)
    wp = jnp.pad(W, ((0, 0), (0, CP - C)))
    b128 = jnp.tile(jnp.pad(b, (0, CP - C)), 128 // CP).reshape(1, 128)

    xwt = _project(x, wp).reshape(CP * N)
    partial = _sc_scatter(xwt, ei4, edge_weight)
    comb = _combine(partial.reshape(NC, FOLD, 128), b128)
    return comb.reshape(N, CP)[:, :C]
